# Initial kernel scaffold; baseline (speedup 1.0000x reference)
import jax
import jax.numpy as jnp
from jax.experimental import pallas as pl


def kernel(x, pe_table):
    raise NotImplementedError("write your pallas kernel here")


# speedup vs baseline: 2.1969x; 2.1969x over previous
import functools

import jax
import jax.numpy as jnp
from jax import lax
from jax.experimental import pallas as pl
from jax.experimental.pallas import tpu as pltpu
from jax.experimental.pallas import tpu_sc as plsc

MAX_LEN = 2048
D_MODEL = 1024
BATCH = 4

_NC = 2
_NS = 16
_NW = _NC * _NS
_ROWS_W = MAX_LEN // _NW


@functools.partial(
    pl.kernel,
    mesh=plsc.VectorSubcoreMesh(core_axis_name="c", subcore_axis_name="s"),
    out_type=jax.ShapeDtypeStruct((BATCH, MAX_LEN, D_MODEL), jnp.float32),
    scratch_types=[
        pltpu.VMEM((_ROWS_W, D_MODEL), jnp.float32),
        pltpu.SemaphoreType.DMA,
    ],
)
def _pe_broadcast(table_hbm, out_hbm, rows_v, sem):
    wid = lax.axis_index("s") * _NC + lax.axis_index("c")
    base = wid * _ROWS_W
    pltpu.sync_copy(table_hbm.at[pl.ds(base, _ROWS_W), :], rows_v)
    copies = [
        pltpu.async_copy(rows_v, out_hbm.at[b, pl.ds(base, _ROWS_W), :], sem)
        for b in range(BATCH)
    ]
    for c in copies:
        c.wait()


def kernel(x, pe_table):
    del x
    return _pe_broadcast(pe_table)
